# baseline (speedup 1.0000x reference)
import functools

import jax
import jax.numpy as jnp
import numpy as np
from jax import lax
from jax.experimental import pallas as pl
from jax.experimental.pallas import tpu as pltpu
from jax.experimental.pallas import tpu_sc as plsc

HID_DIM = 64
N_LAYERS = 2
INPUT_DIM = 32
D = HID_DIM * N_LAYERS
TYPE_NUM = 40
MAX_EDGE_TYPE = 8
N_NODES = 10000
N_EDGES = 320000
SEQ = 32
OP_INPUT = 0
OP_CONST = 1
NG = 6
NONE_G = NG

NP_ = N_NODES
EP = 327680
DW = 144
DWB = 160
NC, NS = 2, 16
NW = NC * NS
EPW = EP // NW
CH = 64
EPC = EP // CH
NCHUNK = EPW // CH
NCHUNK0, NCHUNK1 = 232, 88
NBUF = 2
BKILL = TYPE_NUM * MAX_EDGE_TYPE
NPV = NP_ + 16

_LPERM = np.array([(p // 32) * 32 + (p % 2) * 16 + (p % 32) // 2
                   for p in range(160)], dtype=np.int32)
_INV_LPERM = np.argsort(_LPERM).astype(np.int32)
CHX = 8

BN_K = 1000


def _classify_body(x_ref, io_ref, pv_ref, gs_ref):
    x = x_ref[...]
    col = lax.broadcasted_iota(jnp.int32, (BN_K, TYPE_NUM), 1)
    nt = jnp.max(jnp.where(x == 1.0, col, 0), axis=1, keepdims=True)
    g = jnp.where(
        nt == 2, 0, jnp.where(
            (nt >= 4) & (nt <= 25), 1, jnp.where(
                (nt >= 26) & (nt <= 32), 2, jnp.where(
                    (nt == 34) | (nt == 36), 3, jnp.where(
                        nt == 33, 4, jnp.where(nt == 37, 5, NONE_G))))))
    io_ref[...] = ((nt == OP_INPUT) | (nt == OP_CONST)).astype(jnp.float32)
    pv_ref[...] = jnp.minimum(g, NG - 1) | (nt << 3)
    gs_ref[...] = g


def _classify(x):
    nblk = NP_ // BN_K
    return pl.pallas_call(
        _classify_body,
        grid=(nblk,),
        in_specs=[pl.BlockSpec((BN_K, TYPE_NUM), lambda j: (j, 0))],
        out_specs=[pl.BlockSpec((BN_K, 1), lambda j: (j, 0))] * 3,
        out_shape=[jax.ShapeDtypeStruct((NP_, 1), jnp.float32),
                   jax.ShapeDtypeStruct((NP_, 1), jnp.int32),
                   jax.ShapeDtypeStruct((NP_, 1), jnp.int32)],
    )(x)


BN_A = 1000


def _gru_gates(gi_t, gh, h, hd):
    rz = jax.nn.sigmoid(gi_t[:, 0:2 * hd] + gh[:, 0:2 * hd])
    r = rz[:, 0:hd]
    z = rz[:, hd:2 * hd]
    i_n = gi_t[:, 2 * hd:3 * hd]
    h_n = gh[:, 2 * hd:3 * hd]
    n = jnp.tanh(i_n + r * h_n)
    return (1.0 - z) * n + z * h


def _encode_body(x_ref, sl_ref, io_ref, wi0, wh0, bi0, bh0, wi1, wh1, bi1, bh1,
                 w1p, out_ref, ytab_ref, gi_ref):
    bn = BN_A
    bf16 = jnp.bfloat16
    xm = x_ref[...] * sl_ref[...][:, :, None]
    x2 = xm.reshape(SEQ * bn, INPUT_DIM)
    gi_ref[...] = (jnp.dot(x2, wi0[...], preferred_element_type=jnp.float32)
                   + bi0[...]).reshape(SEQ, bn, 3 * HID_DIM).astype(bf16)

    def step(t, carry):
        h0, h1, y = carry
        gi0_t = gi_ref[jnp.minimum(t, SEQ - 1)].astype(jnp.float32)
        gh0 = jnp.dot(h0.astype(bf16), wh0[...],
                      preferred_element_type=jnp.float32) + bh0[...]
        gi1 = jnp.dot(y.astype(bf16), wi1[...],
                      preferred_element_type=jnp.float32) + bi1[...]
        gh1 = jnp.dot(h1.astype(bf16), wh1[...],
                      preferred_element_type=jnp.float32) + bh1[...]
        h0n = jnp.where(t < SEQ, _gru_gates(gi0_t, gh0, h0, HID_DIM), h0)
        h1n = jnp.where(t >= 1, _gru_gates(gi1, gh1, h1, HID_DIM), h1)
        return (h0n, h1n, h0n)

    z0 = jnp.zeros((bn, HID_DIM), jnp.float32)
    h0, h1, _ = lax.fori_loop(0, SEQ + 1, step, (z0, z0, z0))
    emb = jnp.concatenate([h0, h1], axis=1) * io_ref[...]
    out_ref[...] = emb
    col = lax.broadcasted_iota(jnp.int32, (bn, DWB), 1)
    deg1 = jnp.where(col == D, 1.0, 0.0)
    for g in range(NG):
        ytab_ref[g] = (jnp.dot(emb, w1p[g],
                               preferred_element_type=jnp.float32)
                       + deg1).astype(bf16)


def _encode(xs_tm, slm, io, enc_w, w1pad):
    nblk = NP_ // BN_A
    full = lambda shape: pl.BlockSpec(shape, lambda j: tuple(0 for _ in shape))
    return pl.pallas_call(
        _encode_body,
        grid=(nblk,),
        in_specs=[
            pl.BlockSpec((SEQ, BN_A, INPUT_DIM), lambda j: (0, j, 0)),
            full((SEQ, 1)),
            pl.BlockSpec((BN_A, 1), lambda j: (j, 0)),
            full((INPUT_DIM, 3 * HID_DIM)),
            full((HID_DIM, 3 * HID_DIM)),
            full((1, 3 * HID_DIM)),
            full((1, 3 * HID_DIM)),
            full((HID_DIM, 3 * HID_DIM)),
            full((HID_DIM, 3 * HID_DIM)),
            full((1, 3 * HID_DIM)),
            full((1, 3 * HID_DIM)),
            full((NG, D, DWB)),
        ],
        out_specs=[pl.BlockSpec((BN_A, D), lambda j: (j, 0)),
                   pl.BlockSpec((NG, BN_A, DWB), lambda j: (0, j, 0))],
        out_shape=[jax.ShapeDtypeStruct((NP_, D), jnp.float32),
                   jax.ShapeDtypeStruct((NG, NP_, DWB), jnp.bfloat16)],
        scratch_shapes=[
            pltpu.VMEM((SEQ, BN_A, 3 * HID_DIM), jnp.bfloat16),
        ],
        compiler_params=pltpu.CompilerParams(
            vmem_limit_bytes=63 * 1024 * 1024),
    )(xs_tm, slm, io, *enc_w, w1pad)


def _ixmap_body(pv_hbm, epack, ipack, pvt, ein, eout):
    cid = lax.axis_index("c")
    sid = lax.axis_index("s")
    wid = cid * NS + sid
    pltpu.sync_copy(pv_hbm, pvt)
    base = wid * (NCHUNK // CHX)

    def step(s, c):
        pltpu.sync_copy(epack.at[pl.ds((base + s) * CHX, CHX)], ein)
        for j in range(CHX):
            for k in range(CH // 16):
                sl = pl.ds(k * 16, 16)
                srcv = ein[j, 0, sl]
                dstv = ein[j, 1, sl]
                etv = ein[j, 2, sl]
                pvv = plsc.load_gather(pvt, [dstv])
                g0 = pvv & 7
                ibv = ((pvv >> 3) << 3) + etv
                eout[j, 0, sl] = g0 * NP_ + srcv
                eout[j, 1, sl] = ibv
                eout[j, 2, sl] = jnp.where(dstv < NP_, dstv, srcv)
        pltpu.sync_copy(eout, ipack.at[pl.ds((base + s) * CHX, CHX)])
        return c

    lax.fori_loop(0, NCHUNK // CHX, step, 0)


@functools.cache
def _ixmap_kernel_fn():
    return pl.kernel(
        _ixmap_body,
        mesh=plsc.VectorSubcoreMesh(core_axis_name="c", subcore_axis_name="s",
                                    num_cores=NC, num_subcores=NS),
        out_type=jax.ShapeDtypeStruct((EPC, 3, CH), jnp.int32),
        scratch_types=[
            pltpu.VMEM((NPV,), jnp.int32),
            pltpu.VMEM((CHX, 3, CH), jnp.int32),
            pltpu.VMEM((CHX, 3, CH), jnp.int32),
        ],
        compiler_params=pltpu.CompilerParams(use_tc_tiling_on_sc=False,
                                             needs_layout_passes=False),
    )


def _edge_body(ytab, btab, ipack, out,
               h_sh, pk0, pk1, ry0, ry1, rb0, rb1, rc, sem0, sem1):
    cid = lax.axis_index("c")
    sid = lax.axis_index("s")
    wid = cid * NS + sid
    pk = (pk0, pk1)
    ry = (ry0, ry1)
    rb = (rb0, rb1)
    sem = (sem0, sem1)

    def zrow(r, c):
        for j in range(DW // 16):
            rc[r, pl.ds(j * 16, 16)] = jnp.zeros((16,), jnp.float32)
        return c

    lax.fori_loop(0, CH, zrow, 0)
    rows_per_tile = NP_ // NS
    for k in range(rows_per_tile // CH):
        pltpu.sync_copy(rc, h_sh.at[pl.ds(sid * rows_per_tile + k * CH, CH)])
    rem = rows_per_tile % CH
    if rem:
        pltpu.sync_copy(
            rc.at[pl.ds(0, rem)],
            h_sh.at[pl.ds(sid * rows_per_tile + (rows_per_tile // CH) * CH,
                          rem)])
    plsc.subcore_barrier()

    base_chunk = jnp.where(cid == 0, sid * NCHUNK0,
                           NS * NCHUNK0 + sid * NCHUNK1)
    nchunk = jnp.where(cid == 0, NCHUNK0, NCHUNK1)

    def issue(b, chunk_ix):
        pltpu.sync_copy(ipack.at[chunk_ix], pk[b])
        pltpu.async_copy(ytab.at[pk[b].at[0]], ry[b], sem[b])
        pltpu.async_copy(btab.at[pk[b].at[1]], rb[b], sem[b])

    for b in range(NBUF):
        issue(b, base_chunk + b)

    def pair(jj, c):
        for b in range(NBUF):
            i = jj * NBUF + b
            pltpu.make_async_copy(ytab.at[pk[b].at[0]], ry[b], sem[b]).wait()
            pltpu.make_async_copy(btab.at[pk[b].at[1]], rb[b], sem[b]).wait()

            def row(r, c2, _b=b):
                for j in range(DW // 32):
                    sl = pl.ds(j * 32, 32)
                    v = jnp.maximum(ry[_b][r, sl] + rb[_b][r, sl],
                                    jnp.zeros((32,), jnp.bfloat16))
                    lo, hi = plsc.unpack(v,
                                         format=plsc.PackFormat.INTERLEAVED)
                    rc[r, pl.ds(j * 32, 16)] = lo
                    rc[r, pl.ds(j * 32 + 16, 16)] = hi
                v = jnp.maximum(ry[_b][r, pl.ds(128, 32)]
                                + rb[_b][r, pl.ds(128, 32)],
                                jnp.zeros((32,), jnp.bfloat16))
                lo, hi = plsc.unpack(v, format=plsc.PackFormat.INTERLEAVED)
                rc[r, pl.ds(128, 16)] = lo
                return c2

            lax.fori_loop(0, CH, row, 0)
            pltpu.sync_copy(rc, h_sh.at[pk[b].at[2]], add=True)

            @pl.when(i + NBUF < nchunk)
            def _(b=b, i=i):
                issue(b, base_chunk + i + NBUF)
        return c

    lax.fori_loop(0, nchunk // NBUF, pair, 0)
    plsc.subcore_barrier()
    pltpu.sync_copy(h_sh.at[pl.ds(sid * rows_per_tile, rows_per_tile)],
                    out.at[cid, pl.ds(sid * rows_per_tile, rows_per_tile)])


@functools.cache
def _edge_kernel_fn():
    return pl.kernel(
        _edge_body,
        mesh=plsc.VectorSubcoreMesh(core_axis_name="c", subcore_axis_name="s",
                                    num_cores=NC, num_subcores=NS),
        out_type=jax.ShapeDtypeStruct((NC, NP_, DW), jnp.float32),
        scratch_types=[
            pltpu.VMEM_SHARED((NP_, DW), jnp.float32),
            pltpu.VMEM((3, CH), jnp.int32),
            pltpu.VMEM((3, CH), jnp.int32),
            pltpu.VMEM((CH, DWB), jnp.bfloat16),
            pltpu.VMEM((CH, DWB), jnp.bfloat16),
            pltpu.VMEM((CH, DWB), jnp.bfloat16),
            pltpu.VMEM((CH, DWB), jnp.bfloat16),
            pltpu.VMEM((CH, DW), jnp.float32),
            pltpu.SemaphoreType.DMA,
            pltpu.SemaphoreType.DMA,
        ],
        compiler_params=pltpu.CompilerParams(use_tc_tiling_on_sc=False,
                                             needs_layout_passes=False),
    )


BN_D = 1000


def _out_body(hp_ref, gsel_ref, emb_ref, w2_ref, gwi, gwh, gbi, gbh, out_ref):
    H = hp_ref[0] + hp_ref[1]
    gsel = gsel_ref[...]
    acc = jnp.zeros((BN_D, D), jnp.float32)
    for g in range(NG):
        og = jnp.dot(H, w2_ref[g], preferred_element_type=jnp.float32)
        acc = acc + og * (gsel == g).astype(jnp.float32)
    emb = emb_ref[...]
    gi = jnp.dot(acc, gwi[...], preferred_element_type=jnp.float32) + gbi[...]
    gh = jnp.dot(emb, gwh[...], preferred_element_type=jnp.float32) + gbh[...]
    out_ref[...] = _gru_gates(gi, gh, emb, D)


def _out_gru(hp, gsel, emb, w2aug, gru_w):
    nblk = NP_ // BN_D
    full = lambda shape: pl.BlockSpec(shape, lambda j: tuple(0 for _ in shape))
    return pl.pallas_call(
        _out_body,
        grid=(nblk,),
        in_specs=[
            pl.BlockSpec((NC, BN_D, DW), lambda j: (0, j, 0)),
            pl.BlockSpec((BN_D, 1), lambda j: (j, 0)),
            pl.BlockSpec((BN_D, D), lambda j: (j, 0)),
            full((NG, DW, D)),
            full((D, 3 * D)),
            full((D, 3 * D)),
            full((1, 3 * D)),
            full((1, 3 * D)),
        ],
        out_specs=pl.BlockSpec((BN_D, D), lambda j: (j, 0)),
        out_shape=jax.ShapeDtypeStruct((NP_, D), jnp.float32),
    )(hp, gsel, emb, w2aug, *gru_w)


GROUP_NAMES = ['Reg', 'Bop', 'Sop', 'Mop', 'Cond', 'Output']
_LUT_NP = np.full((TYPE_NUM,), NONE_G, dtype=np.int32)
for _gi, _ids in enumerate([np.array([2]), np.arange(4, 26), np.arange(26, 33),
                            np.array([34, 36]), np.array([33]),
                            np.array([37])]):
    _LUT_NP[_ids] = _gi


def kernel(x, sim_res, params, edge_index, edge_type, seq_len):
    f32 = jnp.float32
    p = params

    io, pv, gsel = _classify(x)

    bf16 = jnp.bfloat16
    xs_tm = jnp.transpose(sim_res.astype(bf16), (1, 0, 2))
    slm = (jnp.arange(SEQ) < seq_len).astype(bf16)[:, None]

    enc_w = (p['enc_Wi_0'].T.astype(bf16), p['enc_Wh_0'].T.astype(bf16),
             p['enc_bi_0'][None, :], p['enc_bh_0'][None, :],
             p['enc_Wi_1'].T.astype(bf16), p['enc_Wh_1'].T.astype(bf16),
             p['enc_bi_1'][None, :], p['enc_bh_1'][None, :])

    W1 = jnp.stack([p[g + '_W1'] for g in GROUP_NAMES])
    b1 = jnp.stack([p[g + '_b1'] for g in GROUP_NAMES])
    W2 = jnp.stack([p[g + '_W2'] for g in GROUP_NAMES])
    b2 = jnp.stack([p[g + '_b2'] for g in GROUP_NAMES])
    pos = jnp.stack([p[g + '_pos'] for g in GROUP_NAMES])
    op = jnp.stack([p[g + '_op'] for g in GROUP_NAMES])

    w1pad = jnp.pad(W1, ((0, 0), (0, 0), (0, DWB - D)))
    w1pad = w1pad[:, :, jnp.asarray(_LPERM)]
    emb, ytab6 = _encode(xs_tm, slm, io, enc_w, w1pad)
    ytab = ytab6.reshape(NG * NP_, DWB)

    lut0 = jnp.minimum(jnp.asarray(_LUT_NP), NG - 1)
    gnt = lut0[jnp.arange(TYPE_NUM)]
    msg_b = pos[gnt] + op[gnt, jnp.arange(TYPE_NUM)][:, None, :]
    btab = (jnp.einsum('ned,ndk->nek', msg_b, W1[gnt])
            + b1[gnt][:, None, :]).reshape(BKILL, D)
    btab = jnp.pad(btab, ((0, 0), (0, DWB - D)))
    btab = jnp.concatenate(
        [btab, jnp.full((1, DWB), -1e30, f32),
         jnp.zeros((7, DWB), f32)])[:, jnp.asarray(_LPERM)].astype(
             jnp.bfloat16)

    pad_e = EP - N_EDGES
    src = jnp.concatenate([edge_index[0].astype(jnp.int32),
                           jnp.arange(pad_e, dtype=jnp.int32) % NP_])
    dst = jnp.concatenate([edge_index[1].astype(jnp.int32),
                           jnp.full((pad_e,), NP_, jnp.int32)])
    et = jnp.concatenate([edge_type.astype(jnp.int32),
                          jnp.zeros((pad_e,), jnp.int32)])
    epack = jnp.stack([src.reshape(EPC, CH), dst.reshape(EPC, CH),
                       et.reshape(EPC, CH)], axis=1)
    pv_full = jnp.concatenate([pv.reshape(-1),
                               jnp.full((NPV - NP_,), TYPE_NUM << 3,
                                        jnp.int32)])

    ipack = _ixmap_kernel_fn()(pv_full, epack)
    hp = _edge_kernel_fn()(ytab, btab, ipack)

    w2aug = jnp.concatenate(
        [W2, b2[:, None, :], jnp.zeros((NG, DW - D - 1, D), f32)], axis=1)
    gru_w = (p['gru_Wi'].T, p['gru_Wh'].T, p['gru_bi'][None, :],
             p['gru_bh'][None, :])
    return _out_gru(hp, gsel, emb, w2aug, gru_w)


# speedup vs baseline: 1.2959x; 1.2959x over previous
import functools

import jax
import jax.numpy as jnp
import numpy as np
from jax import lax
from jax.experimental import pallas as pl
from jax.experimental.pallas import tpu as pltpu
from jax.experimental.pallas import tpu_sc as plsc

HID_DIM = 64
N_LAYERS = 2
INPUT_DIM = 32
D = HID_DIM * N_LAYERS
TYPE_NUM = 40
MAX_EDGE_TYPE = 8
N_NODES = 10000
N_EDGES = 320000
SEQ = 32
OP_INPUT = 0
OP_CONST = 1
NG = 6
NONE_G = NG

NP_ = N_NODES
EP = 327680
DW = 144
NC, NS = 2, 16
NW = NC * NS
EPW = EP // NW
CH = 64
EPC = EP // CH
NCHUNK = EPW // CH
NCHUNK0, NCHUNK1 = 232, 88
NBUF = 2
BKILL = TYPE_NUM * MAX_EDGE_TYPE
NPV = NP_ + 16
CHX = 8

BN_K = 1000


def _classify_body(x_ref, io_ref, pv_ref, gs_ref):
    x = x_ref[...]
    col = lax.broadcasted_iota(jnp.int32, (BN_K, TYPE_NUM), 1)
    nt = jnp.max(jnp.where(x == 1.0, col, 0), axis=1, keepdims=True)
    g = jnp.where(
        nt == 2, 0, jnp.where(
            (nt >= 4) & (nt <= 25), 1, jnp.where(
                (nt >= 26) & (nt <= 32), 2, jnp.where(
                    (nt == 34) | (nt == 36), 3, jnp.where(
                        nt == 33, 4, jnp.where(nt == 37, 5, NONE_G))))))
    io_ref[...] = ((nt == OP_INPUT) | (nt == OP_CONST)).astype(jnp.float32)
    pv_ref[...] = jnp.minimum(g, NG - 1) | (nt << 3)
    gs_ref[...] = g


def _classify(x):
    nblk = NP_ // BN_K
    return pl.pallas_call(
        _classify_body,
        grid=(nblk,),
        in_specs=[pl.BlockSpec((BN_K, TYPE_NUM), lambda j: (j, 0))],
        out_specs=[pl.BlockSpec((BN_K, 1), lambda j: (j, 0))] * 3,
        out_shape=[jax.ShapeDtypeStruct((NP_, 1), jnp.float32),
                   jax.ShapeDtypeStruct((NP_, 1), jnp.int32),
                   jax.ShapeDtypeStruct((NP_, 1), jnp.int32)],
    )(x)


BN_A = 1000


def _gru_gates(gi_t, gh, h, hd):
    rz = jax.nn.sigmoid(gi_t[:, 0:2 * hd] + gh[:, 0:2 * hd])
    r = rz[:, 0:hd]
    z = rz[:, hd:2 * hd]
    i_n = gi_t[:, 2 * hd:3 * hd]
    h_n = gh[:, 2 * hd:3 * hd]
    n = jnp.tanh(i_n + r * h_n)
    return (1.0 - z) * n + z * h


def _encode_body(x_ref, sl_ref, io_ref, wi0, wh0, bi0, bh0, wi1, wh1, bi1, bh1,
                 w1p, out_ref, ytab_ref, gi_ref):
    bn = BN_A
    bf16 = jnp.bfloat16
    xm = x_ref[...] * sl_ref[...][:, :, None]
    x2 = xm.reshape(SEQ * bn, INPUT_DIM)
    gi_ref[...] = (jnp.dot(x2, wi0[...], preferred_element_type=jnp.float32)
                   + bi0[...]).reshape(SEQ, bn, 3 * HID_DIM).astype(bf16)

    def step(t, carry):
        h0, h1, y = carry
        gi0_t = gi_ref[jnp.minimum(t, SEQ - 1)].astype(jnp.float32)
        gh0 = jnp.dot(h0.astype(bf16), wh0[...],
                      preferred_element_type=jnp.float32) + bh0[...]
        gi1 = jnp.dot(y.astype(bf16), wi1[...],
                      preferred_element_type=jnp.float32) + bi1[...]
        gh1 = jnp.dot(h1.astype(bf16), wh1[...],
                      preferred_element_type=jnp.float32) + bh1[...]
        h0n = jnp.where(t < SEQ, _gru_gates(gi0_t, gh0, h0, HID_DIM), h0)
        h1n = jnp.where(t >= 1, _gru_gates(gi1, gh1, h1, HID_DIM), h1)
        return (h0n, h1n, h0n)

    z0 = jnp.zeros((bn, HID_DIM), jnp.float32)
    h0, h1, _ = lax.fori_loop(0, SEQ + 1, step, (z0, z0, z0))
    emb = jnp.concatenate([h0, h1], axis=1) * io_ref[...]
    out_ref[...] = emb
    col = lax.broadcasted_iota(jnp.int32, (bn, DW), 1)
    deg1 = jnp.where(col == D, 1.0, 0.0)
    for g in range(NG):
        ytab_ref[g] = jnp.dot(emb, w1p[g],
                              preferred_element_type=jnp.float32) + deg1


def _encode(xs_tm, slm, io, enc_w, w1pad):
    nblk = NP_ // BN_A
    full = lambda shape: pl.BlockSpec(shape, lambda j: tuple(0 for _ in shape))
    return pl.pallas_call(
        _encode_body,
        grid=(nblk,),
        in_specs=[
            pl.BlockSpec((SEQ, BN_A, INPUT_DIM), lambda j: (0, j, 0)),
            full((SEQ, 1)),
            pl.BlockSpec((BN_A, 1), lambda j: (j, 0)),
            full((INPUT_DIM, 3 * HID_DIM)),
            full((HID_DIM, 3 * HID_DIM)),
            full((1, 3 * HID_DIM)),
            full((1, 3 * HID_DIM)),
            full((HID_DIM, 3 * HID_DIM)),
            full((HID_DIM, 3 * HID_DIM)),
            full((1, 3 * HID_DIM)),
            full((1, 3 * HID_DIM)),
            full((NG, D, DW)),
        ],
        out_specs=[pl.BlockSpec((BN_A, D), lambda j: (j, 0)),
                   pl.BlockSpec((NG, BN_A, DW), lambda j: (0, j, 0))],
        out_shape=[jax.ShapeDtypeStruct((NP_, D), jnp.float32),
                   jax.ShapeDtypeStruct((NG, NP_, DW), jnp.float32)],
        scratch_shapes=[
            pltpu.VMEM((SEQ, BN_A, 3 * HID_DIM), jnp.bfloat16),
        ],
        compiler_params=pltpu.CompilerParams(
            vmem_limit_bytes=63 * 1024 * 1024),
    )(xs_tm, slm, io, *enc_w, w1pad)


def _ixmap_body(pv_hbm, epack, ipack, pvt, ein, eout):
    cid = lax.axis_index("c")
    sid = lax.axis_index("s")
    wid = cid * NS + sid
    pltpu.sync_copy(pv_hbm, pvt)
    base = wid * (NCHUNK // CHX)

    def step(s, c):
        pltpu.sync_copy(epack.at[pl.ds((base + s) * CHX, CHX)], ein)
        for j in range(CHX):
            for k in range(CH // 16):
                sl = pl.ds(k * 16, 16)
                srcv = ein[j, 0, sl]
                dstv = ein[j, 1, sl]
                etv = ein[j, 2, sl]
                pvv = plsc.load_gather(pvt, [dstv])
                g0 = pvv & 7
                ibv = ((pvv >> 3) << 3) + etv
                eout[j, 0, sl] = g0 * NP_ + srcv
                eout[j, 1, sl] = ibv
                eout[j, 2, sl] = jnp.where(dstv < NP_, dstv, srcv)
        pltpu.sync_copy(eout, ipack.at[pl.ds((base + s) * CHX, CHX)])
        return c

    lax.fori_loop(0, NCHUNK // CHX, step, 0)


@functools.cache
def _ixmap_kernel_fn():
    return pl.kernel(
        _ixmap_body,
        mesh=plsc.VectorSubcoreMesh(core_axis_name="c", subcore_axis_name="s",
                                    num_cores=NC, num_subcores=NS),
        out_type=jax.ShapeDtypeStruct((EPC, 3, CH), jnp.int32),
        scratch_types=[
            pltpu.VMEM((NPV,), jnp.int32),
            pltpu.VMEM((CHX, 3, CH), jnp.int32),
            pltpu.VMEM((CHX, 3, CH), jnp.int32),
        ],
        compiler_params=pltpu.CompilerParams(use_tc_tiling_on_sc=False,
                                             needs_layout_passes=False),
    )


def _edge_body(ytab, btab, ipack, out,
               h_sh, pk0, pk1, ry0, ry1, rb0, rb1, sem0, sem1):
    cid = lax.axis_index("c")
    sid = lax.axis_index("s")
    wid = cid * NS + sid
    pk = (pk0, pk1)
    ry = (ry0, ry1)
    rb = (rb0, rb1)
    sem = (sem0, sem1)

    def zrow(r, c):
        for j in range(DW // 16):
            ry0[r, pl.ds(j * 16, 16)] = jnp.zeros((16,), jnp.float32)
        return c

    lax.fori_loop(0, CH, zrow, 0)
    rows_per_tile = NP_ // NS
    for k in range(rows_per_tile // CH):
        pltpu.sync_copy(ry0, h_sh.at[pl.ds(sid * rows_per_tile + k * CH, CH)])
    rem = rows_per_tile % CH
    if rem:
        pltpu.sync_copy(
            ry0.at[pl.ds(0, rem)],
            h_sh.at[pl.ds(sid * rows_per_tile + (rows_per_tile // CH) * CH,
                          rem)])
    plsc.subcore_barrier()

    base_chunk = jnp.where(cid == 0, sid * NCHUNK0,
                           NS * NCHUNK0 + sid * NCHUNK1)
    nchunk = jnp.where(cid == 0, NCHUNK0, NCHUNK1)

    def issue(b, chunk_ix):
        pltpu.sync_copy(ipack.at[chunk_ix], pk[b])
        pltpu.async_copy(ytab.at[pk[b].at[0]], ry[b], sem[b])
        pltpu.async_copy(btab.at[pk[b].at[1]], rb[b], sem[b])

    for b in range(NBUF):
        issue(b, base_chunk + b)

    def pair(jj, c):
        for b in range(NBUF):
            i = jj * NBUF + b
            pltpu.make_async_copy(ytab.at[pk[b].at[0]], ry[b], sem[b]).wait()
            pltpu.make_async_copy(btab.at[pk[b].at[1]], rb[b], sem[b]).wait()

            def row(r, c2, _b=b):
                for j in range(DW // 16):
                    sl = pl.ds(j * 16, 16)
                    ry[_b][r, sl] = jnp.maximum(ry[_b][r, sl] + rb[_b][r, sl],
                                                0.0)
                return c2

            lax.fori_loop(0, CH, row, 0)
            pltpu.sync_copy(ry[b], h_sh.at[pk[b].at[2]], add=True)

            @pl.when(i + NBUF < nchunk)
            def _(b=b, i=i):
                issue(b, base_chunk + i + NBUF)
        return c

    lax.fori_loop(0, nchunk // NBUF, pair, 0)
    plsc.subcore_barrier()
    pltpu.sync_copy(h_sh.at[pl.ds(sid * rows_per_tile, rows_per_tile)],
                    out.at[cid, pl.ds(sid * rows_per_tile, rows_per_tile)])


@functools.cache
def _edge_kernel_fn():
    return pl.kernel(
        _edge_body,
        mesh=plsc.VectorSubcoreMesh(core_axis_name="c", subcore_axis_name="s",
                                    num_cores=NC, num_subcores=NS),
        out_type=jax.ShapeDtypeStruct((NC, NP_, DW), jnp.float32),
        scratch_types=[
            pltpu.VMEM_SHARED((NP_, DW), jnp.float32),
            pltpu.VMEM((3, CH), jnp.int32),
            pltpu.VMEM((3, CH), jnp.int32),
            pltpu.VMEM((CH, DW), jnp.float32),
            pltpu.VMEM((CH, DW), jnp.float32),
            pltpu.VMEM((CH, DW), jnp.float32),
            pltpu.VMEM((CH, DW), jnp.float32),
            pltpu.SemaphoreType.DMA,
            pltpu.SemaphoreType.DMA,
        ],
        compiler_params=pltpu.CompilerParams(use_tc_tiling_on_sc=False),
    )


BN_D = 1000


def _out_body(hp_ref, gsel_ref, emb_ref, w2_ref, gwi, gwh, gbi, gbh, out_ref):
    H = hp_ref[0] + hp_ref[1]
    gsel = gsel_ref[...]
    acc = jnp.zeros((BN_D, D), jnp.float32)
    for g in range(NG):
        og = jnp.dot(H, w2_ref[g], preferred_element_type=jnp.float32)
        acc = acc + og * (gsel == g).astype(jnp.float32)
    emb = emb_ref[...]
    gi = jnp.dot(acc, gwi[...], preferred_element_type=jnp.float32) + gbi[...]
    gh = jnp.dot(emb, gwh[...], preferred_element_type=jnp.float32) + gbh[...]
    out_ref[...] = _gru_gates(gi, gh, emb, D)


def _out_gru(hp, gsel, emb, w2aug, gru_w):
    nblk = NP_ // BN_D
    full = lambda shape: pl.BlockSpec(shape, lambda j: tuple(0 for _ in shape))
    return pl.pallas_call(
        _out_body,
        grid=(nblk,),
        in_specs=[
            pl.BlockSpec((NC, BN_D, DW), lambda j: (0, j, 0)),
            pl.BlockSpec((BN_D, 1), lambda j: (j, 0)),
            pl.BlockSpec((BN_D, D), lambda j: (j, 0)),
            full((NG, DW, D)),
            full((D, 3 * D)),
            full((D, 3 * D)),
            full((1, 3 * D)),
            full((1, 3 * D)),
        ],
        out_specs=pl.BlockSpec((BN_D, D), lambda j: (j, 0)),
        out_shape=jax.ShapeDtypeStruct((NP_, D), jnp.float32),
    )(hp, gsel, emb, w2aug, *gru_w)


GROUP_NAMES = ['Reg', 'Bop', 'Sop', 'Mop', 'Cond', 'Output']
_LUT_NP = np.full((TYPE_NUM,), NONE_G, dtype=np.int32)
for _gi, _ids in enumerate([np.array([2]), np.arange(4, 26), np.arange(26, 33),
                            np.array([34, 36]), np.array([33]),
                            np.array([37])]):
    _LUT_NP[_ids] = _gi


def kernel(x, sim_res, params, edge_index, edge_type, seq_len):
    f32 = jnp.float32
    p = params

    io, pv, gsel = _classify(x)

    bf16 = jnp.bfloat16
    xs_tm = jnp.transpose(sim_res.astype(bf16), (1, 0, 2))
    slm = (jnp.arange(SEQ) < seq_len).astype(bf16)[:, None]

    enc_w = (p['enc_Wi_0'].T.astype(bf16), p['enc_Wh_0'].T.astype(bf16),
             p['enc_bi_0'][None, :], p['enc_bh_0'][None, :],
             p['enc_Wi_1'].T.astype(bf16), p['enc_Wh_1'].T.astype(bf16),
             p['enc_bi_1'][None, :], p['enc_bh_1'][None, :])

    W1 = jnp.stack([p[g + '_W1'] for g in GROUP_NAMES])
    b1 = jnp.stack([p[g + '_b1'] for g in GROUP_NAMES])
    W2 = jnp.stack([p[g + '_W2'] for g in GROUP_NAMES])
    b2 = jnp.stack([p[g + '_b2'] for g in GROUP_NAMES])
    pos = jnp.stack([p[g + '_pos'] for g in GROUP_NAMES])
    op = jnp.stack([p[g + '_op'] for g in GROUP_NAMES])

    w1pad = jnp.pad(W1, ((0, 0), (0, 0), (0, DW - D)))
    emb, ytab6 = _encode(xs_tm, slm, io, enc_w, w1pad)
    ytab = ytab6.reshape(NG * NP_, DW)

    lut0 = jnp.minimum(jnp.asarray(_LUT_NP), NG - 1)
    gnt = lut0[jnp.arange(TYPE_NUM)]
    msg_b = pos[gnt] + op[gnt, jnp.arange(TYPE_NUM)][:, None, :]
    btab = (jnp.einsum('ned,ndk->nek', msg_b, W1[gnt])
            + b1[gnt][:, None, :]).reshape(BKILL, D)
    btab = jnp.pad(btab, ((0, 0), (0, DW - D)))
    btab = jnp.concatenate(
        [btab, jnp.full((1, DW), -1e30, f32), jnp.zeros((7, DW), f32)])

    pad_e = EP - N_EDGES
    src = jnp.concatenate([edge_index[0].astype(jnp.int32),
                           jnp.arange(pad_e, dtype=jnp.int32) % NP_])
    dst = jnp.concatenate([edge_index[1].astype(jnp.int32),
                           jnp.full((pad_e,), NP_, jnp.int32)])
    et = jnp.concatenate([edge_type.astype(jnp.int32),
                          jnp.zeros((pad_e,), jnp.int32)])
    epack = jnp.stack([src.reshape(EPC, CH), dst.reshape(EPC, CH),
                       et.reshape(EPC, CH)], axis=1)
    pv_full = jnp.concatenate([pv.reshape(-1),
                               jnp.full((NPV - NP_,), TYPE_NUM << 3,
                                        jnp.int32)])

    ipack = _ixmap_kernel_fn()(pv_full, epack)
    hp = _edge_kernel_fn()(ytab, btab, ipack)

    w2aug = jnp.concatenate(
        [W2, b2[:, None, :], jnp.zeros((NG, DW - D - 1, D), f32)], axis=1)
    gru_w = (p['gru_Wi'].T, p['gru_Wh'].T, p['gru_bi'][None, :],
             p['gru_bh'][None, :])
    return _out_gru(hp, gsel, emb, w2aug, gru_w)


# speedup vs baseline: 1.3643x; 1.0528x over previous
import functools

import jax
import jax.numpy as jnp
import numpy as np
from jax import lax
from jax.experimental import pallas as pl
from jax.experimental.pallas import tpu as pltpu
from jax.experimental.pallas import tpu_sc as plsc

HID_DIM = 64
N_LAYERS = 2
INPUT_DIM = 32
D = HID_DIM * N_LAYERS
TYPE_NUM = 40
MAX_EDGE_TYPE = 8
N_NODES = 10000
N_EDGES = 320000
SEQ = 32
OP_INPUT = 0
OP_CONST = 1
NG = 6
NONE_G = NG

NP_ = N_NODES
EP = 327680
DW = 144
NC, NS = 2, 16
NW = NC * NS
EPW = EP // NW
CH = 64
EPC = EP // CH
NCHUNK = EPW // CH
NCHUNK0, NCHUNK1 = 232, 88
NBUF = 2
BKILL = TYPE_NUM * MAX_EDGE_TYPE
NPV = NP_ + 16
CHX = 8

BN_K = 1000


def _classify_body(x_ref, io_ref, pv_ref, gs_ref):
    x = x_ref[...]
    col = lax.broadcasted_iota(jnp.int32, (BN_K, TYPE_NUM), 1)
    nt = jnp.max(jnp.where(x == 1.0, col, 0), axis=1, keepdims=True)
    g = jnp.where(
        nt == 2, 0, jnp.where(
            (nt >= 4) & (nt <= 25), 1, jnp.where(
                (nt >= 26) & (nt <= 32), 2, jnp.where(
                    (nt == 34) | (nt == 36), 3, jnp.where(
                        nt == 33, 4, jnp.where(nt == 37, 5, NONE_G))))))
    io_ref[...] = ((nt == OP_INPUT) | (nt == OP_CONST)).astype(jnp.float32)
    pv_ref[...] = jnp.minimum(g, NG - 1) | (nt << 3)
    gs_ref[...] = g


def _classify(x):
    nblk = NP_ // BN_K
    return pl.pallas_call(
        _classify_body,
        grid=(nblk,),
        in_specs=[pl.BlockSpec((BN_K, TYPE_NUM), lambda j: (j, 0))],
        out_specs=[pl.BlockSpec((BN_K, 1), lambda j: (j, 0))] * 3,
        out_shape=[jax.ShapeDtypeStruct((NP_, 1), jnp.float32),
                   jax.ShapeDtypeStruct((NP_, 1), jnp.int32),
                   jax.ShapeDtypeStruct((NP_, 1), jnp.int32)],
    )(x)


BN_A = 1000


def _gru_gates(gi_t, gh, h, hd):
    rz = jax.nn.sigmoid(gi_t[:, 0:2 * hd] + gh[:, 0:2 * hd])
    r = rz[:, 0:hd]
    z = rz[:, hd:2 * hd]
    i_n = gi_t[:, 2 * hd:3 * hd]
    h_n = gh[:, 2 * hd:3 * hd]
    n = jnp.tanh(i_n + r * h_n)
    return (1.0 - z) * n + z * h


def _encode_body(x_ref, sl_ref, io_ref, wi0, wh0, bi0, bh0, wi1, wh1, bi1, bh1,
                 w1p, out_ref, ytab_ref, gi_ref):
    bn = BN_A
    bf16 = jnp.bfloat16
    xm = x_ref[...] * sl_ref[...][:, :, None]
    x2 = xm.reshape(SEQ * bn, INPUT_DIM)
    gi_ref[...] = (jnp.dot(x2, wi0[...], preferred_element_type=jnp.float32)
                   + bi0[...]).reshape(SEQ, bn, 3 * HID_DIM).astype(bf16)

    def step(t, carry):
        h0, h1, y = carry
        gi0_t = gi_ref[jnp.minimum(t, SEQ - 1)].astype(jnp.float32)
        gh0 = jnp.dot(h0.astype(bf16), wh0[...],
                      preferred_element_type=jnp.float32) + bh0[...]
        gi1 = jnp.dot(y.astype(bf16), wi1[...],
                      preferred_element_type=jnp.float32) + bi1[...]
        gh1 = jnp.dot(h1.astype(bf16), wh1[...],
                      preferred_element_type=jnp.float32) + bh1[...]
        h0n = jnp.where(t < SEQ, _gru_gates(gi0_t, gh0, h0, HID_DIM), h0)
        h1n = jnp.where(t >= 1, _gru_gates(gi1, gh1, h1, HID_DIM), h1)
        return (h0n, h1n, h0n)

    z0 = jnp.zeros((bn, HID_DIM), jnp.float32)
    h0, h1, _ = lax.fori_loop(0, SEQ + 1, step, (z0, z0, z0))
    emb = jnp.concatenate([h0, h1], axis=1) * io_ref[...]
    out_ref[...] = emb
    col = lax.broadcasted_iota(jnp.int32, (bn, DW), 1)
    deg1 = jnp.where(col == D, 1.0, 0.0)
    for g in range(NG):
        ytab_ref[g] = jnp.dot(emb, w1p[g],
                              preferred_element_type=jnp.float32) + deg1


def _encode(xs_tm, slm, io, enc_w, w1pad):
    nblk = NP_ // BN_A
    full = lambda shape: pl.BlockSpec(shape, lambda j: tuple(0 for _ in shape))
    return pl.pallas_call(
        _encode_body,
        grid=(nblk,),
        in_specs=[
            pl.BlockSpec((SEQ, BN_A, INPUT_DIM), lambda j: (0, j, 0)),
            full((SEQ, 1)),
            pl.BlockSpec((BN_A, 1), lambda j: (j, 0)),
            full((INPUT_DIM, 3 * HID_DIM)),
            full((HID_DIM, 3 * HID_DIM)),
            full((1, 3 * HID_DIM)),
            full((1, 3 * HID_DIM)),
            full((HID_DIM, 3 * HID_DIM)),
            full((HID_DIM, 3 * HID_DIM)),
            full((1, 3 * HID_DIM)),
            full((1, 3 * HID_DIM)),
            full((NG, D, DW)),
        ],
        out_specs=[pl.BlockSpec((BN_A, D), lambda j: (j, 0)),
                   pl.BlockSpec((NG, BN_A, DW), lambda j: (0, j, 0))],
        out_shape=[jax.ShapeDtypeStruct((NP_, D), jnp.float32),
                   jax.ShapeDtypeStruct((NG, NP_, DW), jnp.float32)],
        scratch_shapes=[
            pltpu.VMEM((SEQ, BN_A, 3 * HID_DIM), jnp.bfloat16),
        ],
        compiler_params=pltpu.CompilerParams(
            vmem_limit_bytes=63 * 1024 * 1024),
    )(xs_tm, slm, io, *enc_w, w1pad)


def _ixmap_body(pv_hbm, src_hbm, dst_hbm, et_hbm, ipack,
                pvt, es, ed, ee, eout):
    cid = lax.axis_index("c")
    sid = lax.axis_index("s")
    wid = cid * NS + sid
    pltpu.sync_copy(pv_hbm, pvt)
    base = wid * (NCHUNK // CHX)
    sc = CHX * CH

    def step(s, c):
        pltpu.sync_copy(src_hbm.at[pl.ds((base + s) * sc, sc)], es)
        pltpu.sync_copy(dst_hbm.at[pl.ds((base + s) * sc, sc)], ed)
        pltpu.sync_copy(et_hbm.at[pl.ds((base + s) * sc, sc)], ee)
        for k in range(sc // 16):
            sl = pl.ds(k * 16, 16)
            j, o = k // (CH // 16), (k % (CH // 16)) * 16
            so = pl.ds(o, 16)
            srcv = es[sl]
            dstv = ed[sl]
            etv = ee[sl]
            pvv = plsc.load_gather(pvt, [dstv])
            g0 = pvv & 7
            ibv = ((pvv >> 3) << 3) + etv
            eout[j, 0, so] = g0 * NP_ + srcv
            eout[j, 1, so] = ibv
            eout[j, 2, so] = jnp.where(dstv < NP_, dstv, srcv)
        pltpu.sync_copy(eout, ipack.at[pl.ds((base + s) * CHX, CHX)])
        return c

    lax.fori_loop(0, NCHUNK // CHX, step, 0)


@functools.cache
def _ixmap_kernel_fn():
    return pl.kernel(
        _ixmap_body,
        mesh=plsc.VectorSubcoreMesh(core_axis_name="c", subcore_axis_name="s",
                                    num_cores=NC, num_subcores=NS),
        out_type=jax.ShapeDtypeStruct((EPC, 3, CH), jnp.int32),
        scratch_types=[
            pltpu.VMEM((NPV,), jnp.int32),
            pltpu.VMEM((CHX * CH,), jnp.int32),
            pltpu.VMEM((CHX * CH,), jnp.int32),
            pltpu.VMEM((CHX * CH,), jnp.int32),
            pltpu.VMEM((CHX, 3, CH), jnp.int32),
        ],
        compiler_params=pltpu.CompilerParams(use_tc_tiling_on_sc=False,
                                             needs_layout_passes=False),
    )


def _edge_body(ytab, btab, ipack, out,
               h_sh, pk0, pk1, ry0, ry1, rb0, rb1, sem0, sem1):
    cid = lax.axis_index("c")
    sid = lax.axis_index("s")
    wid = cid * NS + sid
    pk = (pk0, pk1)
    ry = (ry0, ry1)
    rb = (rb0, rb1)
    sem = (sem0, sem1)

    def zrow(r, c):
        for j in range(DW // 16):
            ry0[r, pl.ds(j * 16, 16)] = jnp.zeros((16,), jnp.float32)
        return c

    lax.fori_loop(0, CH, zrow, 0)
    rows_per_tile = NP_ // NS
    for k in range(rows_per_tile // CH):
        pltpu.sync_copy(ry0, h_sh.at[pl.ds(sid * rows_per_tile + k * CH, CH)])
    rem = rows_per_tile % CH
    if rem:
        pltpu.sync_copy(
            ry0.at[pl.ds(0, rem)],
            h_sh.at[pl.ds(sid * rows_per_tile + (rows_per_tile // CH) * CH,
                          rem)])
    plsc.subcore_barrier()

    base_chunk = jnp.where(cid == 0, sid * NCHUNK0,
                           NS * NCHUNK0 + sid * NCHUNK1)
    nchunk = jnp.where(cid == 0, NCHUNK0, NCHUNK1)

    def issue(b, chunk_ix):
        pltpu.sync_copy(ipack.at[chunk_ix], pk[b])
        pltpu.async_copy(ytab.at[pk[b].at[0]], ry[b], sem[b])
        pltpu.async_copy(btab.at[pk[b].at[1]], rb[b], sem[b])

    for b in range(NBUF):
        issue(b, base_chunk + b)

    def pair(jj, c):
        for b in range(NBUF):
            i = jj * NBUF + b
            pltpu.make_async_copy(ytab.at[pk[b].at[0]], ry[b], sem[b]).wait()
            pltpu.make_async_copy(btab.at[pk[b].at[1]], rb[b], sem[b]).wait()

            def row(r, c2, _b=b):
                for j in range(DW // 16):
                    sl = pl.ds(j * 16, 16)
                    ry[_b][r, sl] = jnp.maximum(ry[_b][r, sl] + rb[_b][r, sl],
                                                0.0)
                return c2

            lax.fori_loop(0, CH, row, 0)
            pltpu.sync_copy(ry[b], h_sh.at[pk[b].at[2]], add=True)

            @pl.when(i + NBUF < nchunk)
            def _(b=b, i=i):
                issue(b, base_chunk + i + NBUF)
        return c

    lax.fori_loop(0, nchunk // NBUF, pair, 0)
    plsc.subcore_barrier()
    pltpu.sync_copy(h_sh.at[pl.ds(sid * rows_per_tile, rows_per_tile)],
                    out.at[cid, pl.ds(sid * rows_per_tile, rows_per_tile)])


@functools.cache
def _edge_kernel_fn():
    return pl.kernel(
        _edge_body,
        mesh=plsc.VectorSubcoreMesh(core_axis_name="c", subcore_axis_name="s",
                                    num_cores=NC, num_subcores=NS),
        out_type=jax.ShapeDtypeStruct((NC, NP_, DW), jnp.float32),
        scratch_types=[
            pltpu.VMEM_SHARED((NP_, DW), jnp.float32),
            pltpu.VMEM((3, CH), jnp.int32),
            pltpu.VMEM((3, CH), jnp.int32),
            pltpu.VMEM((CH, DW), jnp.float32),
            pltpu.VMEM((CH, DW), jnp.float32),
            pltpu.VMEM((CH, DW), jnp.float32),
            pltpu.VMEM((CH, DW), jnp.float32),
            pltpu.SemaphoreType.DMA,
            pltpu.SemaphoreType.DMA,
        ],
        compiler_params=pltpu.CompilerParams(use_tc_tiling_on_sc=False),
    )


BN_D = 1000


def _out_body(hp_ref, gsel_ref, emb_ref, w2_ref, gwi, gwh, gbi, gbh, out_ref):
    H = hp_ref[0] + hp_ref[1]
    gsel = gsel_ref[...]
    acc = jnp.zeros((BN_D, D), jnp.float32)
    for g in range(NG):
        og = jnp.dot(H, w2_ref[g], preferred_element_type=jnp.float32)
        acc = acc + og * (gsel == g).astype(jnp.float32)
    emb = emb_ref[...]
    gi = jnp.dot(acc, gwi[...], preferred_element_type=jnp.float32) + gbi[...]
    gh = jnp.dot(emb, gwh[...], preferred_element_type=jnp.float32) + gbh[...]
    out_ref[...] = _gru_gates(gi, gh, emb, D)


def _out_gru(hp, gsel, emb, w2aug, gru_w):
    nblk = NP_ // BN_D
    full = lambda shape: pl.BlockSpec(shape, lambda j: tuple(0 for _ in shape))
    return pl.pallas_call(
        _out_body,
        grid=(nblk,),
        in_specs=[
            pl.BlockSpec((NC, BN_D, DW), lambda j: (0, j, 0)),
            pl.BlockSpec((BN_D, 1), lambda j: (j, 0)),
            pl.BlockSpec((BN_D, D), lambda j: (j, 0)),
            full((NG, DW, D)),
            full((D, 3 * D)),
            full((D, 3 * D)),
            full((1, 3 * D)),
            full((1, 3 * D)),
        ],
        out_specs=pl.BlockSpec((BN_D, D), lambda j: (j, 0)),
        out_shape=jax.ShapeDtypeStruct((NP_, D), jnp.float32),
    )(hp, gsel, emb, w2aug, *gru_w)


GROUP_NAMES = ['Reg', 'Bop', 'Sop', 'Mop', 'Cond', 'Output']
_LUT_NP = np.full((TYPE_NUM,), NONE_G, dtype=np.int32)
for _gi, _ids in enumerate([np.array([2]), np.arange(4, 26), np.arange(26, 33),
                            np.array([34, 36]), np.array([33]),
                            np.array([37])]):
    _LUT_NP[_ids] = _gi


def kernel(x, sim_res, params, edge_index, edge_type, seq_len):
    f32 = jnp.float32
    p = params

    io, pv, gsel = _classify(x)

    bf16 = jnp.bfloat16
    xs_tm = jnp.transpose(sim_res.astype(bf16), (1, 0, 2))
    slm = (jnp.arange(SEQ) < seq_len).astype(bf16)[:, None]

    enc_w = (p['enc_Wi_0'].T.astype(bf16), p['enc_Wh_0'].T.astype(bf16),
             p['enc_bi_0'][None, :], p['enc_bh_0'][None, :],
             p['enc_Wi_1'].T.astype(bf16), p['enc_Wh_1'].T.astype(bf16),
             p['enc_bi_1'][None, :], p['enc_bh_1'][None, :])

    W1 = jnp.stack([p[g + '_W1'] for g in GROUP_NAMES])
    b1 = jnp.stack([p[g + '_b1'] for g in GROUP_NAMES])
    W2 = jnp.stack([p[g + '_W2'] for g in GROUP_NAMES])
    b2 = jnp.stack([p[g + '_b2'] for g in GROUP_NAMES])
    pos = jnp.stack([p[g + '_pos'] for g in GROUP_NAMES])
    op = jnp.stack([p[g + '_op'] for g in GROUP_NAMES])

    w1pad = jnp.pad(W1, ((0, 0), (0, 0), (0, DW - D)))
    emb, ytab6 = _encode(xs_tm, slm, io, enc_w, w1pad)
    ytab = ytab6.reshape(NG * NP_, DW)

    lut0 = jnp.minimum(jnp.asarray(_LUT_NP), NG - 1)
    gnt = lut0[jnp.arange(TYPE_NUM)]
    msg_b = pos[gnt] + op[gnt, jnp.arange(TYPE_NUM)][:, None, :]
    btab = (jnp.einsum('ned,ndk->nek', msg_b, W1[gnt])
            + b1[gnt][:, None, :]).reshape(BKILL, D)
    btab = jnp.pad(btab, ((0, 0), (0, DW - D)))
    btab = jnp.concatenate(
        [btab, jnp.full((1, DW), -1e30, f32), jnp.zeros((7, DW), f32)])

    pad_e = EP - N_EDGES
    src = jnp.concatenate([edge_index[0].astype(jnp.int32),
                           jnp.arange(pad_e, dtype=jnp.int32) % NP_])
    dst = jnp.concatenate([edge_index[1].astype(jnp.int32),
                           jnp.full((pad_e,), NP_, jnp.int32)])
    et = jnp.concatenate([edge_type.astype(jnp.int32),
                          jnp.zeros((pad_e,), jnp.int32)])
    pv_full = jnp.concatenate([pv.reshape(-1),
                               jnp.full((NPV - NP_,), TYPE_NUM << 3,
                                        jnp.int32)])

    ipack = _ixmap_kernel_fn()(pv_full, src, dst, et)
    hp = _edge_kernel_fn()(ytab, btab, ipack)

    w2aug = jnp.concatenate(
        [W2, b2[:, None, :], jnp.zeros((NG, DW - D - 1, D), f32)], axis=1)
    gru_w = (p['gru_Wi'].T, p['gru_Wh'].T, p['gru_bi'][None, :],
             p['gru_bh'][None, :])
    return _out_gru(hp, gsel, emb, w2aug, gru_w)
